# Initial kernel scaffold; baseline (speedup 1.0000x reference)
import jax
import jax.numpy as jnp
from jax.experimental import pallas as pl


def kernel(inputs, targets, class_weights):
    raise NotImplementedError("write your pallas kernel here")


# speedup vs baseline: 4.6191x; 4.6191x over previous
import functools

import jax
import jax.numpy as jnp
from jax.experimental import pallas as pl
from jax.experimental.pallas import tpu as pltpu

_VMEM_LIMIT_BYTES = 64 * 1024 * 1024


def _wce_body(x_ref, t_ref, w_ref, loss_ref, wsum_ref, *, n_classes):
    j = pl.program_id(1)

    @pl.when(j == 0)
    def _():
        loss_ref[...] = jnp.zeros_like(loss_ref)
        wsum_ref[...] = jnp.zeros_like(wsum_ref)

    t = t_ref[0, 0]

    m = x_ref[0, 0]
    for c in range(1, n_classes):
        m = jnp.maximum(m, x_ref[0, c])
    s = jnp.exp(x_ref[0, 0] - m)
    for c in range(1, n_classes):
        s = s + jnp.exp(x_ref[0, c] - m)
    lse = m + jnp.log(s)

    logit_t = jnp.zeros_like(m)
    w_t = jnp.zeros_like(m)
    for c in range(n_classes):
        hit = t == c
        logit_t = jnp.where(hit, x_ref[0, c], logit_t)
        w_t = jnp.where(hit, w_ref[c], w_t)

    loss_ref[0] += w_t * (lse - logit_t)
    wsum_ref[0] += w_t


def kernel(inputs, targets, class_weights):
    n, c, h, w = inputs.shape
    t4 = targets.reshape(n, 1, h, w)
    cw = class_weights.astype(jnp.float32)

    n_cores = 2 if n % 2 == 0 else 1
    n_per_core = n // n_cores

    def in_map(p, j):
        return (p * n_per_core + j, 0, 0, 0)

    loss_p, wsum_p = pl.pallas_call(
        functools.partial(_wce_body, n_classes=c),
        grid=(n_cores, n_per_core),
        in_specs=[pl.BlockSpec((1, c, h, w), in_map),
                  pl.BlockSpec((1, 1, h, w), in_map),
                  pl.BlockSpec(memory_space=pltpu.MemorySpace.SMEM)],
        out_specs=[pl.BlockSpec((1, h, w), lambda p, j: (p, 0, 0)),
                   pl.BlockSpec((1, h, w), lambda p, j: (p, 0, 0))],
        out_shape=(jax.ShapeDtypeStruct((n_cores, h, w), jnp.float32),
                   jax.ShapeDtypeStruct((n_cores, h, w), jnp.float32)),
        compiler_params=pltpu.CompilerParams(
            dimension_semantics=("parallel", "arbitrary"),
            vmem_limit_bytes=_VMEM_LIMIT_BYTES),
    )(inputs, t4, cw)
    return jnp.sum(loss_p) / jnp.sum(wsum_p)


# speedup vs baseline: 7.1707x; 1.5524x over previous
import functools

import jax
import jax.numpy as jnp
from jax.experimental import pallas as pl
from jax.experimental.pallas import tpu as pltpu

_VMEM_LIMIT_BYTES = 64 * 1024 * 1024


def _wce_body(x_ref, t_ref, w_ref, o_ref, lacc_ref, wacc_ref, *,
              n_classes, n_steps, block_n):
    j = pl.program_id(0)

    @pl.when(j == 0)
    def _():
        lacc_ref[...] = jnp.zeros_like(lacc_ref)
        wacc_ref[...] = jnp.zeros_like(wacc_ref)

    for b in range(block_n):
        t = t_ref[b, 0]

        m = x_ref[b, 0]
        for c in range(1, n_classes):
            m = jnp.maximum(m, x_ref[b, c])
        s = jnp.exp(x_ref[b, 0] - m)
        for c in range(1, n_classes):
            s = s + jnp.exp(x_ref[b, c] - m)
        lse = m + jnp.log(s)

        logit_t = jnp.zeros_like(m)
        w_t = jnp.zeros_like(m)
        for c in range(n_classes):
            hit = t == c
            logit_t = jnp.where(hit, x_ref[b, c], logit_t)
            w_t = jnp.where(hit, w_ref[c], w_t)

        lacc_ref[...] += w_t * (lse - logit_t)
        wacc_ref[...] += w_t

    @pl.when(j == n_steps - 1)
    def _():
        o_ref[0, 0] = jnp.sum(lacc_ref[...]) / jnp.sum(wacc_ref[...])


def kernel(inputs, targets, class_weights):
    n, c, h, w = inputs.shape
    t4 = targets.reshape(n, 1, h, w)
    cw = class_weights.astype(jnp.float32)

    block_n = 2 if n % 2 == 0 else 1
    n_steps = n // block_n

    out = pl.pallas_call(
        functools.partial(_wce_body, n_classes=c, n_steps=n_steps,
                          block_n=block_n),
        grid=(n_steps,),
        in_specs=[pl.BlockSpec((block_n, c, h, w), lambda j: (j, 0, 0, 0)),
                  pl.BlockSpec((block_n, 1, h, w), lambda j: (j, 0, 0, 0)),
                  pl.BlockSpec(memory_space=pltpu.MemorySpace.SMEM)],
        out_specs=pl.BlockSpec(memory_space=pltpu.MemorySpace.SMEM),
        out_shape=jax.ShapeDtypeStruct((1, 1), jnp.float32),
        scratch_shapes=[pltpu.VMEM((h, w), jnp.float32),
                        pltpu.VMEM((h, w), jnp.float32)],
        compiler_params=pltpu.CompilerParams(
            dimension_semantics=("arbitrary",),
            vmem_limit_bytes=_VMEM_LIMIT_BYTES),
    )(inputs, t4, cw)
    return out[0, 0]


# speedup vs baseline: 7.5685x; 1.0555x over previous
import functools

import jax
import jax.numpy as jnp
from jax.experimental import pallas as pl
from jax.experimental.pallas import tpu as pltpu

_VMEM_LIMIT_BYTES = 64 * 1024 * 1024


def _wce_body(x_ref, t_ref, w_ref, o_ref, lacc_ref, wacc_ref, *,
              n_classes, n_steps, block_n):
    j = pl.program_id(0)

    @pl.when(j == 0)
    def _():
        lacc_ref[...] = jnp.zeros_like(lacc_ref)
        wacc_ref[...] = jnp.zeros_like(wacc_ref)

    for b in range(block_n):
        t = t_ref[b, 0]

        m = x_ref[b, 0]
        for c in range(1, n_classes):
            m = jnp.maximum(m, x_ref[b, c])
        s = jnp.exp(x_ref[b, 0] - m)
        for c in range(1, n_classes):
            s = s + jnp.exp(x_ref[b, c] - m)
        lse = m + jnp.log(s)

        logit_t = jnp.zeros_like(m)
        w_t = jnp.zeros_like(m)
        for c in range(n_classes):
            hit = t == c
            logit_t = jnp.where(hit, x_ref[b, c], logit_t)
            w_t = jnp.where(hit, w_ref[c], w_t)

        lacc_ref[...] += w_t * (lse - logit_t)
        wacc_ref[...] += w_t

    @pl.when(j == n_steps - 1)
    def _():
        o_ref[0, 0] = jnp.sum(lacc_ref[...]) / jnp.sum(wacc_ref[...])


def kernel(inputs, targets, class_weights):
    n, c, h, w = inputs.shape
    t4 = targets.reshape(n, 1, h, w)
    cw = class_weights.astype(jnp.float32)

    block_n = 4 if n % 4 == 0 else (2 if n % 2 == 0 else 1)
    n_steps = n // block_n

    out = pl.pallas_call(
        functools.partial(_wce_body, n_classes=c, n_steps=n_steps,
                          block_n=block_n),
        grid=(n_steps,),
        in_specs=[pl.BlockSpec((block_n, c, h, w), lambda j: (j, 0, 0, 0)),
                  pl.BlockSpec((block_n, 1, h, w), lambda j: (j, 0, 0, 0)),
                  pl.BlockSpec(memory_space=pltpu.MemorySpace.SMEM)],
        out_specs=pl.BlockSpec(memory_space=pltpu.MemorySpace.SMEM),
        out_shape=jax.ShapeDtypeStruct((1, 1), jnp.float32),
        scratch_shapes=[pltpu.VMEM((h, w), jnp.float32),
                        pltpu.VMEM((h, w), jnp.float32)],
        compiler_params=pltpu.CompilerParams(
            dimension_semantics=("arbitrary",),
            vmem_limit_bytes=_VMEM_LIMIT_BYTES),
    )(inputs, t4, cw)
    return out[0, 0]
